# baseline (speedup 1.0000x reference)
import functools

import jax
import jax.numpy as jnp
from jax import lax
from jax.experimental import pallas as pl
from jax.experimental.pallas import tpu as pltpu
from jax.experimental.pallas import tpu_sc as plsc

N = 10000
E = 160000
NPAD = 10112
RPT = NPAD // 16
CH = 112
WCH = 90
NCH = 45
EPAD = 16 * WCH * CH
NBUF = 6
LOOK = 3
BLK = 2000
BBLK = 5000
EBLK = 16000


def _gs_pipe(nc, issue_g, wait_g, issue_s, wait_s):
    dd = NBUF - LOOK
    for q in range(min(LOOK, nc)):
        issue_g(q, q % NBUF)
    full = (nc // NBUF) * NBUF

    @pl.loop(0, full, step=NBUF)
    def _(t):
        for b in range(NBUF):
            j = t + b
            wait_g(j, b)
            issue_s(j, b)
            nj = j + LOOK
            nb = (b + LOOK) % NBUF

            @pl.when(jnp.logical_and(nj < nc, j >= dd))
            def _():
                wait_s(j - dd, nb)

            @pl.when(nj < nc)
            def _():
                issue_g(nj, nb)

    for j in range(full, nc):
        b = j % NBUF
        wait_g(j, b)
        issue_s(j, b)
    for j in range(max(0, nc - NBUF), nc):
        wait_s(j, j % NBUF)


@functools.cache
def _sc_kernels():
    mesh = plsc.VectorSubcoreMesh(core_axis_name="c", subcore_axis_name="s",
                                  num_cores=2, num_subcores=16)
    dma = pltpu.SemaphoreType.DMA
    sems = [dma] * (2 * NBUF)

    @functools.partial(
        pl.kernel,
        out_type=(
            jax.ShapeDtypeStruct((NPAD, 128), jnp.bfloat16),
            jax.ShapeDtypeStruct((NPAD, 128), jnp.bfloat16),
        ),
        mesh=mesh,
        scratch_types=[
            pltpu.VMEM((WCH, CH), jnp.int32),
            pltpu.VMEM((WCH, CH), jnp.int32),
            pltpu.VMEM((NBUF, CH, 128), jnp.bfloat16),
            pltpu.VMEM_SHARED((NPAD, 128), jnp.bfloat16),
        ] + sems,
        compiler_params=pltpu.CompilerParams(use_tc_tiling_on_sc=False),
    )
    def _wide_agg(u0, u1, srcw, dstw, zz, s0, s1, src_v, dst_v, buf, acc,
                  *sm):
        cid = lax.axis_index("c")
        sid = lax.axis_index("s")
        z0 = sid * RPT
        gs, ss = sm[:NBUF], sm[NBUF:2 * NBUF]
        pltpu.sync_copy(zz.at[pl.ds(z0, RPT)], acc.at[pl.ds(z0, RPT)])
        pltpu.sync_copy(srcw.at[sid], src_v)
        pltpu.sync_copy(dstw.at[sid], dst_v)
        plsc.subcore_barrier()

        def run(tbl, out):
            _gs_pipe(
                WCH,
                lambda j, b: pltpu.async_copy(
                    tbl.at[src_v.at[j]], buf.at[b], gs[b]),
                lambda j, b: pltpu.make_async_copy(
                    tbl.at[src_v.at[j]], buf.at[b], gs[b]).wait(),
                lambda j, b: pltpu.async_copy(
                    buf.at[b], acc.at[dst_v.at[j]], ss[b], add=True),
                lambda j, b: pltpu.make_async_copy(
                    buf.at[b], acc.at[dst_v.at[j]], ss[b]).wait(),
            )
            plsc.subcore_barrier()
            pltpu.sync_copy(acc.at[pl.ds(z0, RPT)], out.at[pl.ds(z0, RPT)])

        @pl.when(cid == 0)
        def _():
            run(u0, s0)

        @pl.when(cid == 1)
        def _():
            run(u1, s1)

    def _make_narrow(w):
        @functools.partial(
            pl.kernel,
            out_type=jax.ShapeDtypeStruct((2, NPAD, w), jnp.float32),
            mesh=mesh,
            scratch_types=[
                pltpu.VMEM((NCH, CH), jnp.int32),
                pltpu.VMEM((NCH, CH), jnp.int32),
                pltpu.VMEM((NBUF, CH, w), jnp.float32),
                pltpu.VMEM_SHARED((NPAD, w), jnp.float32),
            ] + sems,
            compiler_params=pltpu.CompilerParams(use_tc_tiling_on_sc=False),
        )
        def _narrow_agg(u, srcn, dstn, zz, outp, src_v, dst_v, buf, acc,
                        *sm):
            cid = lax.axis_index("c")
            sid = lax.axis_index("s")
            z0 = sid * RPT
            gs, ss = sm[:NBUF], sm[NBUF:2 * NBUF]
            pltpu.sync_copy(zz.at[pl.ds(z0, RPT)], acc.at[pl.ds(z0, RPT)])
            pltpu.sync_copy(srcn.at[sid, cid], src_v)
            pltpu.sync_copy(dstn.at[sid, cid], dst_v)
            plsc.subcore_barrier()

            _gs_pipe(
                NCH,
                lambda j, b: pltpu.async_copy(
                    u.at[src_v.at[j]], buf.at[b], gs[b]),
                lambda j, b: pltpu.make_async_copy(
                    u.at[src_v.at[j]], buf.at[b], gs[b]).wait(),
                lambda j, b: pltpu.async_copy(
                    buf.at[b], acc.at[dst_v.at[j]], ss[b], add=True),
                lambda j, b: pltpu.make_async_copy(
                    buf.at[b], acc.at[dst_v.at[j]], ss[b]).wait(),
            )
            plsc.subcore_barrier()
            pltpu.sync_copy(acc.at[pl.ds(z0, RPT)],
                            outp.at[cid, pl.ds(z0, RPT)])

        return _narrow_agg

    @functools.partial(
        pl.kernel,
        out_type=jax.ShapeDtypeStruct((2, NPAD, 16), jnp.float32),
        mesh=mesh,
        scratch_types=[
            pltpu.VMEM((NCH, CH), jnp.int32),
            pltpu.VMEM((CH, 16), jnp.float32),
            pltpu.VMEM_SHARED((NPAD, 16), jnp.float32),
        ] + [dma] * NBUF,
        compiler_params=pltpu.CompilerParams(use_tc_tiling_on_sc=False),
    )
    def _deg_kernel(dstn, ones, zz, outp, dst_v, ones_v, acc, *sm):
        cid = lax.axis_index("c")
        sid = lax.axis_index("s")
        z0 = sid * RPT
        ss = sm[:NBUF]
        pltpu.sync_copy(zz.at[pl.ds(z0, RPT)], acc.at[pl.ds(z0, RPT)])
        pltpu.sync_copy(dstn.at[sid, cid], dst_v)
        pltpu.sync_copy(ones, ones_v)
        plsc.subcore_barrier()

        full = (NCH // NBUF) * NBUF

        @pl.loop(0, full, step=NBUF)
        def _(t):
            for b in range(NBUF):
                j = t + b

                @pl.when(j >= NBUF)
                def _():
                    pltpu.make_async_copy(
                        ones_v, acc.at[dst_v.at[j - NBUF]], ss[b]).wait()

                pltpu.async_copy(ones_v, acc.at[dst_v.at[j]], ss[b],
                                 add=True)

        for j in range(full, NCH):
            b = j % NBUF
            pltpu.make_async_copy(
                ones_v, acc.at[dst_v.at[j - NBUF]], ss[b]).wait()
            pltpu.async_copy(ones_v, acc.at[dst_v.at[j]], ss[b], add=True)
        for j in range(NCH - NBUF, NCH):
            pltpu.make_async_copy(
                ones_v, acc.at[dst_v.at[j]], ss[j % NBUF]).wait()

        plsc.subcore_barrier()
        pltpu.sync_copy(acc.at[pl.ds(z0, RPT)], outp.at[cid, pl.ds(z0, RPT)])

    @functools.partial(
        pl.kernel,
        out_type=(
            jax.ShapeDtypeStruct((EPAD, 32), jnp.float32),
            jax.ShapeDtypeStruct((EPAD, 32), jnp.float32),
        ),
        mesh=mesh,
        scratch_types=[
            pltpu.VMEM((WCH, CH), jnp.int32),
            pltpu.VMEM((NBUF, CH, 32), jnp.float32),
        ] + sems,
        compiler_params=pltpu.CompilerParams(use_tc_tiling_on_sc=False),
    )
    def _egather(zt, srcw, dstw, zs, zd, idx_v, buf, *sm):
        cid = lax.axis_index("c")
        sid = lax.axis_index("s")
        gs, ss = sm[:NBUF], sm[NBUF:]

        def run(idxa, out):
            pltpu.sync_copy(idxa.at[sid], idx_v)
            _gs_pipe(
                WCH,
                lambda j, b: pltpu.async_copy(
                    zt.at[idx_v.at[j]], buf.at[b], gs[b]),
                lambda j, b: pltpu.make_async_copy(
                    zt.at[idx_v.at[j]], buf.at[b], gs[b]).wait(),
                lambda j, b: pltpu.async_copy(
                    buf.at[b],
                    out.at[pl.ds(sid * (WCH * CH) + j * CH, CH)], ss[b]),
                lambda j, b: pltpu.make_async_copy(
                    buf.at[b],
                    out.at[pl.ds(sid * (WCH * CH) + j * CH, CH)],
                    ss[b]).wait(),
            )

        @pl.when(cid == 0)
        def _():
            run(srcw, zs)

        @pl.when(cid == 1)
        def _():
            run(dstw, zd)

    return {
        "wide": _wide_agg,
        "narrow32": _make_narrow(32),
        "narrow16": _make_narrow(16),
        "deg": _deg_kernel,
        "egather": _egather,
    }


def _mm(a, w):
    return lax.dot_general(a, w, (((1,), (0,)), ((), ())),
                           preferred_element_type=jnp.float32)


def _mmb(a, w):
    return lax.dot_general(a.astype(jnp.bfloat16), w.astype(jnp.bfloat16),
                           (((1,), (0,)), ((), ())),
                           preferred_element_type=jnp.float32)


def _relu(v):
    return jnp.maximum(v, 0.0)


def _rows(w):
    return pl.BlockSpec((BLK, w), lambda i: (i, 0))


def _rowsb(w):
    return pl.BlockSpec((BBLK, w), lambda i: (i, 0))


def _full(shape):
    nd = len(shape)
    return pl.BlockSpec(shape, lambda i: (0,) * nd)


def _out(w, dt=jnp.float32):
    return jax.ShapeDtypeStruct((N, w), dt)


def _degspec():
    return pl.BlockSpec((2, BBLK, 16), lambda i: (0, i, 0))


def _dnorm(degp_blk):
    deg = degp_blk[0, :, 0:1] + degp_blk[1, :, 0:1]
    dmax = jnp.maximum(deg, 1.0)
    conn = jnp.minimum(deg, 1.0)
    return dmax, conn


def _ae_body(x, we1, be1, we2, be2, we3, be3, wz, bz, wd1, bd1, wd2, bd2,
             wd3, bd3, wxb, bxb, wp1, bp1, mu,
             t1o, t2o, t3o, zo, xbo, qo, g1o, g1lo, g1hi):
    xv = x[...]
    t1 = _relu(_mmb(xv, we1[...]) + be1[...])
    t2 = _relu(_mmb(t1, we2[...]) + be2[...])
    t3 = _relu(_mmb(t2, we3[...]) + be3[...])
    zv = _mm(t3, wz[...]) + bz[...]
    d1 = _relu(_mmb(zv, wd1[...]) + bd1[...])
    d2 = _relu(_mmb(d1, wd2[...]) + bd2[...])
    d3 = _relu(_mmb(d2, wd3[...]) + bd3[...])
    xb = _mmb(d3, wxb[...]) + bxb[...]
    g1 = _relu(_mmb(xv, wp1[...]) + bp1[...])
    muv = mu[...]
    z2 = jnp.sum(zv * zv, axis=1, keepdims=True)
    m2 = jnp.sum(muv * muv, axis=1)[None, :]
    cross = lax.dot_general(zv, muv, (((1,), (1,)), ((), ())),
                            preferred_element_type=jnp.float32)
    d2c = z2 - 2.0 * cross + m2
    qu = 1.0 / (1.0 + d2c)
    qv = qu / jnp.sum(qu, axis=1, keepdims=True)
    t1o[...] = t1
    t2o[...] = t2
    t3o[...] = t3
    zo[...] = zv
    xbo[...] = xb
    qo[...] = qv
    g1o[...] = g1
    g16 = g1.astype(jnp.bfloat16)
    g1lo[...] = g16[:, :128]
    g1hi[...] = g16[:, 128:]


def _ae_call(x, p):
    ws = []
    specs = [_rows(128)]
    for nm in ("enc1", "enc2", "enc3", "zl", "dec1", "dec2", "dec3", "xbar",
               "proj1"):
        w = p[nm]["w"]
        b = p[nm]["b"].reshape(1, -1)
        ws += [w, b]
        specs += [_full(w.shape), _full(b.shape)]
    mu = p["cluster"]
    ws.append(mu)
    specs.append(_full(mu.shape))
    return pl.pallas_call(
        _ae_body,
        grid=(N // BLK,),
        in_specs=specs,
        out_specs=[_rows(256), _rows(256), _rows(512), _rows(32), _rows(128),
                   _rows(10), _rows(256), _rows(128), _rows(128)],
        out_shape=[_out(256), _out(256), _out(512), _out(32), _out(128),
                   _out(10), _out(256), _out(128, jnp.bfloat16),
                   _out(128, jnp.bfloat16)],
    )(x, *ws)


def _b1_body(s1lo, s1hi, g1, t1, degp, u2o, u2lo, u2hi):
    dmax, _ = _dnorm(degp[...])
    s1 = jnp.concatenate([s1lo[...], s1hi[...]],
                         axis=1).astype(jnp.float32)
    h1 = _relu(s1 / dmax + g1[...])
    u2 = 0.5 * h1 + 0.5 * t1[...]
    u2o[...] = u2
    u16 = u2.astype(jnp.bfloat16)
    u2lo[...] = u16[:, :128]
    u2hi[...] = u16[:, 128:]


def _b1_call(s1lo, s1hi, g1, t1, degp):
    return pl.pallas_call(
        _b1_body,
        grid=(N // BBLK,),
        in_specs=[_rowsb(128), _rowsb(128), _rowsb(256), _rowsb(256),
                  _degspec()],
        out_specs=[_rowsb(256), _rowsb(128), _rowsb(128)],
        out_shape=[_out(256), _out(128, jnp.bfloat16),
                   _out(128, jnp.bfloat16)],
    )(s1lo, s1hi, g1, t1, degp)


def _b2_body(slo, shi, u, t2, degp, w2, b2, u3o, u3lo, u3hi):
    dmax, conn = _dnorm(degp[...])
    s = jnp.concatenate([slo[...], shi[...]], axis=1).astype(jnp.float32)
    a = s / dmax + u[...]
    h2 = _relu(_mmb(a, w2[...]) + b2[...] * (1.0 + conn))
    u3 = 0.5 * h2 + 0.5 * t2[...]
    u3o[...] = u3
    u16 = u3.astype(jnp.bfloat16)
    u3lo[...] = u16[:, :128]
    u3hi[...] = u16[:, 128:]


def _b2_call(slo, shi, u, t2, degp, p):
    w2 = p["proj2"]["w"]
    b2 = p["proj2"]["b"].reshape(1, -1)
    return pl.pallas_call(
        _b2_body,
        grid=(N // BBLK,),
        in_specs=[_rowsb(128), _rowsb(128), _rowsb(256), _rowsb(256),
                  _degspec(), _full(w2.shape), _full(b2.shape)],
        out_specs=[_rowsb(256), _rowsb(128), _rowsb(128)],
        out_shape=[_out(256), _out(128, jnp.bfloat16),
                   _out(128, jnp.bfloat16)],
    )(slo, shi, u, t2, degp, w2, b2)


def _b3_body(slo, shi, u, t3, degp, w3, b3, w4, b4, g4o):
    dmax, conn = _dnorm(degp[...])
    s = jnp.concatenate([slo[...], shi[...]], axis=1).astype(jnp.float32)
    a = s / dmax + u[...]
    h3 = _relu(_mmb(a, w3[...]) + b3[...] * (1.0 + conn))
    u4 = 0.5 * h3 + 0.5 * t3[...]
    g4o[...] = _mmb(u4, w4[...]) + b4[...]


def _b3_call(slo, shi, u, t3, degp, p):
    w3 = p["proj3"]["w"]
    b3 = p["proj3"]["b"].reshape(1, -1)
    w4 = p["proj4"]["w"]
    b4 = p["proj4"]["b"].reshape(1, -1)
    return pl.pallas_call(
        _b3_body,
        grid=(N // BBLK,),
        in_specs=[_rowsb(128), _rowsb(128), _rowsb(256), _rowsb(512),
                  _degspec(), _full(w3.shape), _full(b3.shape),
                  _full(w4.shape), _full(b4.shape)],
        out_specs=_rowsb(32),
        out_shape=_out(32),
    )(slo, shi, u, t3, degp, w3, b3, w4, b4)


def _b4_body(s4p, g4, z, degp, w5, b5, g5o):
    dmax, _ = _dnorm(degp[...])
    s4 = s4p[0] + s4p[1]
    h4 = _relu(s4 / dmax + g4[...])
    u5 = 0.5 * h4 + 0.5 * z[...]
    g5o[...] = _mm(u5, w5[...]) + b5[...]


def _b4_call(s4p, g4, z, degp, w5, b5):
    return pl.pallas_call(
        _b4_body,
        grid=(N // BBLK,),
        in_specs=[pl.BlockSpec((2, BBLK, 32), lambda i: (0, i, 0)),
                  _rowsb(32), _rowsb(32), _degspec(), _full(w5.shape),
                  _full(b5.shape)],
        out_specs=_rowsb(16),
        out_shape=_out(16),
    )(s4p, g4, z, degp, w5, b5)


def _b5_body(s5p, g5, degp, po):
    dmax, _ = _dnorm(degp[...])
    s5 = s5p[0] + s5p[1]
    h5 = _relu(s5 / dmax + g5[...])
    lg = h5[:, :10]
    m = jnp.max(lg, axis=1, keepdims=True)
    e = jnp.exp(lg - m)
    po[...] = e / jnp.sum(e, axis=1, keepdims=True)


def _b5_call(s5p, g5, degp):
    return pl.pallas_call(
        _b5_body,
        grid=(N // BBLK,),
        in_specs=[pl.BlockSpec((2, BBLK, 16), lambda i: (0, i, 0)),
                  _rowsb(16), _degspec()],
        out_specs=_rowsb(10),
        out_shape=_out(10),
    )(s5p, g5, degp)


def _edec_body(zs, zd, wt, wb, b, po):
    po[...] = _mm(zs[...], wt[...]) + _mm(zd[...], wb[...]) + b[...]


def _edec_call(zs, zd, p):
    w = p["edge_fc"]["w"]
    wt = w[:32]
    wb = w[32:]
    b = p["edge_fc"]["b"].reshape(1, -1)
    espec = pl.BlockSpec((EBLK, 32), lambda i: (i, 0))
    return pl.pallas_call(
        _edec_body,
        grid=(E // EBLK,),
        in_specs=[espec, espec, _full(wt.shape), _full(wb.shape),
                  _full(b.shape)],
        out_specs=pl.BlockSpec((EBLK, 16), lambda i: (i, 0)),
        out_shape=jax.ShapeDtypeStruct((E, 16), jnp.float32),
    )(zs, zd, wt, wb, b)


def kernel(x, edge_attr, params, edge_index):
    p = params
    src = edge_index[0].astype(jnp.int32)
    dst = edge_index[1].astype(jnp.int32)
    pad = EPAD - E
    srcp = jnp.concatenate([src, jnp.zeros((pad,), jnp.int32)])
    dstp = jnp.concatenate([dst, jnp.full((pad,), N, jnp.int32)])
    srcw = srcp.reshape(16, WCH, CH)
    dstw = dstp.reshape(16, WCH, CH)
    srcn = srcp.reshape(16, 2, NCH, CH)
    dstn = dstp.reshape(16, 2, NCH, CH)
    zz128 = jnp.zeros((NPAD, 128), jnp.bfloat16)
    zz32 = jnp.zeros((NPAD, 32), jnp.float32)
    zz16 = jnp.zeros((NPAD, 16), jnp.float32)
    ones = jnp.ones((CH, 16), jnp.float32)

    sc = _sc_kernels()
    t1, t2, t3, z, xbar, q, g1, g1lo, g1hi = _ae_call(x, p)
    degp = sc["deg"](dstn, ones, zz16)

    s1lo, s1hi = sc["wide"](g1lo, g1hi, srcw, dstw, zz128)
    u2, u2lo, u2hi = _b1_call(s1lo, s1hi, g1, t1, degp)
    s2lo, s2hi = sc["wide"](u2lo, u2hi, srcw, dstw, zz128)
    u3, u3lo, u3hi = _b2_call(s2lo, s2hi, u2, t2, degp, p)
    s3lo, s3hi = sc["wide"](u3lo, u3hi, srcw, dstw, zz128)
    g4 = _b3_call(s3lo, s3hi, u3, t3, degp, p)
    s4p = sc["narrow32"](g4, srcn, dstn, zz32)
    w5 = jnp.pad(p["proj5"]["w"], ((0, 0), (0, 6)))
    b5 = jnp.pad(p["proj5"]["b"], (0, 6)).reshape(1, -1)
    g5 = _b4_call(s4p, g4, z, degp, w5, b5)
    s5p = sc["narrow16"](g5, srcn, dstn, zz16)
    predict = _b5_call(s5p, g5, degp)

    zs, zd = sc["egather"](z, srcw, dstw)
    pred_edge = _edec_call(zs, zd, p)
    return (xbar, q, predict, z, pred_edge)


# speedup vs baseline: 1.0168x; 1.0168x over previous
import functools

import jax
import jax.numpy as jnp
from jax import lax
from jax.experimental import pallas as pl
from jax.experimental.pallas import tpu as pltpu
from jax.experimental.pallas import tpu_sc as plsc

N = 10000
E = 160000
NPAD = 10112
RPT = NPAD // 16
CH = 112
WCH = 90
NCH = 45
EPAD = 16 * WCH * CH
NBUF = 6
LOOK = 3
BLK = 2000
BBLK = 2000
EBLK = 16000


def _gs_pipe(nc, issue_g, wait_g, issue_s, wait_s):
    dd = NBUF - LOOK
    for q in range(min(LOOK, nc)):
        issue_g(q, q % NBUF)
    full = (nc // NBUF) * NBUF

    @pl.loop(0, full, step=NBUF)
    def _(t):
        for b in range(NBUF):
            j = t + b
            wait_g(j, b)
            issue_s(j, b)
            nj = j + LOOK
            nb = (b + LOOK) % NBUF

            @pl.when(jnp.logical_and(nj < nc, j >= dd))
            def _():
                wait_s(j - dd, nb)

            @pl.when(nj < nc)
            def _():
                issue_g(nj, nb)

    for j in range(full, nc):
        b = j % NBUF
        wait_g(j, b)
        issue_s(j, b)
    for j in range(max(0, nc - NBUF), nc):
        wait_s(j, j % NBUF)


@functools.cache
def _sc_kernels():
    mesh = plsc.VectorSubcoreMesh(core_axis_name="c", subcore_axis_name="s",
                                  num_cores=2, num_subcores=16)
    dma = pltpu.SemaphoreType.DMA
    sems = [dma] * (2 * NBUF)

    @functools.partial(
        pl.kernel,
        out_type=(
            jax.ShapeDtypeStruct((NPAD, 128), jnp.bfloat16),
            jax.ShapeDtypeStruct((NPAD, 128), jnp.bfloat16),
        ),
        mesh=mesh,
        scratch_types=[
            pltpu.VMEM((WCH, CH), jnp.int32),
            pltpu.VMEM((WCH, CH), jnp.int32),
            pltpu.VMEM((NBUF, CH, 128), jnp.bfloat16),
            pltpu.VMEM_SHARED((NPAD, 128), jnp.bfloat16),
        ] + sems,
        compiler_params=pltpu.CompilerParams(use_tc_tiling_on_sc=False),
    )
    def _wide_agg(u0, u1, srcw, dstw, zz, s0, s1, src_v, dst_v, buf, acc,
                  *sm):
        cid = lax.axis_index("c")
        sid = lax.axis_index("s")
        z0 = sid * RPT
        gs, ss = sm[:NBUF], sm[NBUF:2 * NBUF]
        pltpu.sync_copy(zz.at[pl.ds(z0, RPT)], acc.at[pl.ds(z0, RPT)])
        pltpu.sync_copy(srcw.at[sid], src_v)
        pltpu.sync_copy(dstw.at[sid], dst_v)
        plsc.subcore_barrier()

        def run(tbl, out):
            _gs_pipe(
                WCH,
                lambda j, b: pltpu.async_copy(
                    tbl.at[src_v.at[j]], buf.at[b], gs[b]),
                lambda j, b: pltpu.make_async_copy(
                    tbl.at[src_v.at[j]], buf.at[b], gs[b]).wait(),
                lambda j, b: pltpu.async_copy(
                    buf.at[b], acc.at[dst_v.at[j]], ss[b], add=True),
                lambda j, b: pltpu.make_async_copy(
                    buf.at[b], acc.at[dst_v.at[j]], ss[b]).wait(),
            )
            plsc.subcore_barrier()
            pltpu.sync_copy(acc.at[pl.ds(z0, RPT)], out.at[pl.ds(z0, RPT)])

        @pl.when(cid == 0)
        def _():
            run(u0, s0)

        @pl.when(cid == 1)
        def _():
            run(u1, s1)

    def _make_narrow(w):
        @functools.partial(
            pl.kernel,
            out_type=jax.ShapeDtypeStruct((2, NPAD, w), jnp.float32),
            mesh=mesh,
            scratch_types=[
                pltpu.VMEM((NCH, CH), jnp.int32),
                pltpu.VMEM((NCH, CH), jnp.int32),
                pltpu.VMEM((NBUF, CH, w), jnp.float32),
                pltpu.VMEM_SHARED((NPAD, w), jnp.float32),
            ] + sems,
            compiler_params=pltpu.CompilerParams(use_tc_tiling_on_sc=False),
        )
        def _narrow_agg(u, srcn, dstn, zz, outp, src_v, dst_v, buf, acc,
                        *sm):
            cid = lax.axis_index("c")
            sid = lax.axis_index("s")
            z0 = sid * RPT
            gs, ss = sm[:NBUF], sm[NBUF:2 * NBUF]
            pltpu.sync_copy(zz.at[pl.ds(z0, RPT)], acc.at[pl.ds(z0, RPT)])
            pltpu.sync_copy(srcn.at[sid, cid], src_v)
            pltpu.sync_copy(dstn.at[sid, cid], dst_v)
            plsc.subcore_barrier()

            _gs_pipe(
                NCH,
                lambda j, b: pltpu.async_copy(
                    u.at[src_v.at[j]], buf.at[b], gs[b]),
                lambda j, b: pltpu.make_async_copy(
                    u.at[src_v.at[j]], buf.at[b], gs[b]).wait(),
                lambda j, b: pltpu.async_copy(
                    buf.at[b], acc.at[dst_v.at[j]], ss[b], add=True),
                lambda j, b: pltpu.make_async_copy(
                    buf.at[b], acc.at[dst_v.at[j]], ss[b]).wait(),
            )
            plsc.subcore_barrier()
            pltpu.sync_copy(acc.at[pl.ds(z0, RPT)],
                            outp.at[cid, pl.ds(z0, RPT)])

        return _narrow_agg

    @functools.partial(
        pl.kernel,
        out_type=jax.ShapeDtypeStruct((2, NPAD, 16), jnp.float32),
        mesh=mesh,
        scratch_types=[
            pltpu.VMEM((NCH, CH), jnp.int32),
            pltpu.VMEM((CH, 16), jnp.float32),
            pltpu.VMEM_SHARED((NPAD, 16), jnp.float32),
        ] + [dma] * NBUF,
        compiler_params=pltpu.CompilerParams(use_tc_tiling_on_sc=False),
    )
    def _deg_kernel(dstn, ones, zz, outp, dst_v, ones_v, acc, *sm):
        cid = lax.axis_index("c")
        sid = lax.axis_index("s")
        z0 = sid * RPT
        ss = sm[:NBUF]
        pltpu.sync_copy(zz.at[pl.ds(z0, RPT)], acc.at[pl.ds(z0, RPT)])
        pltpu.sync_copy(dstn.at[sid, cid], dst_v)
        pltpu.sync_copy(ones, ones_v)
        plsc.subcore_barrier()

        full = (NCH // NBUF) * NBUF

        @pl.loop(0, full, step=NBUF)
        def _(t):
            for b in range(NBUF):
                j = t + b

                @pl.when(j >= NBUF)
                def _():
                    pltpu.make_async_copy(
                        ones_v, acc.at[dst_v.at[j - NBUF]], ss[b]).wait()

                pltpu.async_copy(ones_v, acc.at[dst_v.at[j]], ss[b],
                                 add=True)

        for j in range(full, NCH):
            b = j % NBUF
            pltpu.make_async_copy(
                ones_v, acc.at[dst_v.at[j - NBUF]], ss[b]).wait()
            pltpu.async_copy(ones_v, acc.at[dst_v.at[j]], ss[b], add=True)
        for j in range(NCH - NBUF, NCH):
            pltpu.make_async_copy(
                ones_v, acc.at[dst_v.at[j]], ss[j % NBUF]).wait()

        plsc.subcore_barrier()
        pltpu.sync_copy(acc.at[pl.ds(z0, RPT)], outp.at[cid, pl.ds(z0, RPT)])

    @functools.partial(
        pl.kernel,
        out_type=(
            jax.ShapeDtypeStruct((EPAD, 32), jnp.float32),
            jax.ShapeDtypeStruct((EPAD, 32), jnp.float32),
        ),
        mesh=mesh,
        scratch_types=[
            pltpu.VMEM((WCH, CH), jnp.int32),
            pltpu.VMEM((NBUF, CH, 32), jnp.float32),
        ] + sems,
        compiler_params=pltpu.CompilerParams(use_tc_tiling_on_sc=False),
    )
    def _egather(zt, srcw, dstw, zs, zd, idx_v, buf, *sm):
        cid = lax.axis_index("c")
        sid = lax.axis_index("s")
        gs, ss = sm[:NBUF], sm[NBUF:]

        def run(idxa, out):
            pltpu.sync_copy(idxa.at[sid], idx_v)
            _gs_pipe(
                WCH,
                lambda j, b: pltpu.async_copy(
                    zt.at[idx_v.at[j]], buf.at[b], gs[b]),
                lambda j, b: pltpu.make_async_copy(
                    zt.at[idx_v.at[j]], buf.at[b], gs[b]).wait(),
                lambda j, b: pltpu.async_copy(
                    buf.at[b],
                    out.at[pl.ds(sid * (WCH * CH) + j * CH, CH)], ss[b]),
                lambda j, b: pltpu.make_async_copy(
                    buf.at[b],
                    out.at[pl.ds(sid * (WCH * CH) + j * CH, CH)],
                    ss[b]).wait(),
            )

        @pl.when(cid == 0)
        def _():
            run(srcw, zs)

        @pl.when(cid == 1)
        def _():
            run(dstw, zd)

    return {
        "wide": _wide_agg,
        "narrow32": _make_narrow(32),
        "narrow16": _make_narrow(16),
        "deg": _deg_kernel,
        "egather": _egather,
    }


def _mm(a, w):
    return lax.dot_general(a, w, (((1,), (0,)), ((), ())),
                           preferred_element_type=jnp.float32)


def _mmb(a, w):
    return lax.dot_general(a.astype(jnp.bfloat16), w.astype(jnp.bfloat16),
                           (((1,), (0,)), ((), ())),
                           preferred_element_type=jnp.float32)


def _relu(v):
    return jnp.maximum(v, 0.0)


def _rows(w):
    return pl.BlockSpec((BLK, w), lambda i: (i, 0))


def _rowsb(w):
    return pl.BlockSpec((BBLK, w), lambda i: (i, 0))


def _full(shape):
    nd = len(shape)
    return pl.BlockSpec(shape, lambda i: (0,) * nd)


def _out(w, dt=jnp.float32):
    return jax.ShapeDtypeStruct((N, w), dt)


def _degspec():
    return pl.BlockSpec((2, BBLK, 16), lambda i: (0, i, 0))


def _dnorm(degp_blk):
    deg = degp_blk[0, :, 0:1] + degp_blk[1, :, 0:1]
    dmax = jnp.maximum(deg, 1.0)
    conn = jnp.minimum(deg, 1.0)
    return dmax, conn


def _ae_body(x, we1, be1, we2, be2, we3, be3, wz, bz, wd1, bd1, wd2, bd2,
             wd3, bd3, wxb, bxb, wp1, bp1, mu,
             t1o, t2o, t3o, zo, xbo, qo, g1lo, g1hi):
    xv = x[...]
    t1 = _relu(_mmb(xv, we1[...]) + be1[...])
    t2 = _relu(_mmb(t1, we2[...]) + be2[...])
    t3 = _relu(_mmb(t2, we3[...]) + be3[...])
    zv = _mm(t3, wz[...]) + bz[...]
    d1 = _relu(_mmb(zv, wd1[...]) + bd1[...])
    d2 = _relu(_mmb(d1, wd2[...]) + bd2[...])
    d3 = _relu(_mmb(d2, wd3[...]) + bd3[...])
    xb = _mmb(d3, wxb[...]) + bxb[...]
    g1 = _relu(_mmb(xv, wp1[...]) + bp1[...])
    muv = mu[...]
    z2 = jnp.sum(zv * zv, axis=1, keepdims=True)
    m2 = jnp.sum(muv * muv, axis=1)[None, :]
    cross = lax.dot_general(zv, muv, (((1,), (1,)), ((), ())),
                            preferred_element_type=jnp.float32)
    d2c = z2 - 2.0 * cross + m2
    qu = 1.0 / (1.0 + d2c)
    qv = qu / jnp.sum(qu, axis=1, keepdims=True)
    t1o[...] = t1
    t2o[...] = t2
    t3o[...] = t3
    zo[...] = zv
    xbo[...] = xb
    qo[...] = qv
    g16 = g1.astype(jnp.bfloat16)
    g1lo[...] = g16[:, :128]
    g1hi[...] = g16[:, 128:]


def _ae_call(x, p):
    ws = []
    specs = [_rows(128)]
    for nm in ("enc1", "enc2", "enc3", "zl", "dec1", "dec2", "dec3", "xbar",
               "proj1"):
        w = p[nm]["w"]
        b = p[nm]["b"].reshape(1, -1)
        ws += [w, b]
        specs += [_full(w.shape), _full(b.shape)]
    mu = p["cluster"]
    ws.append(mu)
    specs.append(_full(mu.shape))
    return pl.pallas_call(
        _ae_body,
        grid=(N // BLK,),
        in_specs=specs,
        out_specs=[_rows(256), _rows(256), _rows(512), _rows(32), _rows(128),
                   _rows(10), _rows(128), _rows(128)],
        out_shape=[_out(256), _out(256), _out(512), _out(32), _out(128),
                   _out(10), _out(128, jnp.bfloat16),
                   _out(128, jnp.bfloat16)],
    )(x, *ws)


def _b1_body(s1lo, s1hi, g1lo, g1hi, t1, degp, u2lo, u2hi):
    dmax, _ = _dnorm(degp[...])
    s1 = jnp.concatenate([s1lo[...], s1hi[...]],
                         axis=1).astype(jnp.float32)
    g1 = jnp.concatenate([g1lo[...], g1hi[...]],
                         axis=1).astype(jnp.float32)
    h1 = _relu(s1 / dmax + g1)
    u2 = 0.5 * h1 + 0.5 * t1[...]
    u16 = u2.astype(jnp.bfloat16)
    u2lo[...] = u16[:, :128]
    u2hi[...] = u16[:, 128:]


def _b1_call(s1lo, s1hi, g1lo, g1hi, t1, degp):
    return pl.pallas_call(
        _b1_body,
        grid=(N // BBLK,),
        in_specs=[_rowsb(128), _rowsb(128), _rowsb(128), _rowsb(128),
                  _rowsb(256), _degspec()],
        out_specs=[_rowsb(128), _rowsb(128)],
        out_shape=[_out(128, jnp.bfloat16), _out(128, jnp.bfloat16)],
    )(s1lo, s1hi, g1lo, g1hi, t1, degp)


def _b2_body(slo, shi, ulo, uhi, t2, degp, w2, b2, u3lo, u3hi):
    dmax, conn = _dnorm(degp[...])
    s = jnp.concatenate([slo[...], shi[...]], axis=1).astype(jnp.float32)
    u = jnp.concatenate([ulo[...], uhi[...]], axis=1).astype(jnp.float32)
    a = s / dmax + u
    h2 = _relu(_mmb(a, w2[...]) + b2[...] * (1.0 + conn))
    u3 = 0.5 * h2 + 0.5 * t2[...]
    u16 = u3.astype(jnp.bfloat16)
    u3lo[...] = u16[:, :128]
    u3hi[...] = u16[:, 128:]


def _b2_call(slo, shi, ulo, uhi, t2, degp, p):
    w2 = p["proj2"]["w"]
    b2 = p["proj2"]["b"].reshape(1, -1)
    return pl.pallas_call(
        _b2_body,
        grid=(N // BBLK,),
        in_specs=[_rowsb(128), _rowsb(128), _rowsb(128), _rowsb(128),
                  _rowsb(256), _degspec(), _full(w2.shape), _full(b2.shape)],
        out_specs=[_rowsb(128), _rowsb(128)],
        out_shape=[_out(128, jnp.bfloat16), _out(128, jnp.bfloat16)],
    )(slo, shi, ulo, uhi, t2, degp, w2, b2)


def _b3_body(slo, shi, ulo, uhi, t3, degp, w3, b3, w4, b4, g4o):
    dmax, conn = _dnorm(degp[...])
    s = jnp.concatenate([slo[...], shi[...]], axis=1).astype(jnp.float32)
    u = jnp.concatenate([ulo[...], uhi[...]], axis=1).astype(jnp.float32)
    a = s / dmax + u
    h3 = _relu(_mmb(a, w3[...]) + b3[...] * (1.0 + conn))
    u4 = 0.5 * h3 + 0.5 * t3[...]
    g4o[...] = _mmb(u4, w4[...]) + b4[...]


def _b3_call(slo, shi, ulo, uhi, t3, degp, p):
    w3 = p["proj3"]["w"]
    b3 = p["proj3"]["b"].reshape(1, -1)
    w4 = p["proj4"]["w"]
    b4 = p["proj4"]["b"].reshape(1, -1)
    return pl.pallas_call(
        _b3_body,
        grid=(N // BBLK,),
        in_specs=[_rowsb(128), _rowsb(128), _rowsb(128), _rowsb(128),
                  _rowsb(512), _degspec(), _full(w3.shape), _full(b3.shape),
                  _full(w4.shape), _full(b4.shape)],
        out_specs=_rowsb(32),
        out_shape=_out(32),
    )(slo, shi, ulo, uhi, t3, degp, w3, b3, w4, b4)


def _b4_body(s4p, g4, z, degp, w5, b5, g5o):
    dmax, _ = _dnorm(degp[...])
    s4 = s4p[0] + s4p[1]
    h4 = _relu(s4 / dmax + g4[...])
    u5 = 0.5 * h4 + 0.5 * z[...]
    g5o[...] = _mm(u5, w5[...]) + b5[...]


def _b4_call(s4p, g4, z, degp, w5, b5):
    return pl.pallas_call(
        _b4_body,
        grid=(N // BBLK,),
        in_specs=[pl.BlockSpec((2, BBLK, 32), lambda i: (0, i, 0)),
                  _rowsb(32), _rowsb(32), _degspec(), _full(w5.shape),
                  _full(b5.shape)],
        out_specs=_rowsb(16),
        out_shape=_out(16),
    )(s4p, g4, z, degp, w5, b5)


def _b5_body(s5p, g5, degp, po):
    dmax, _ = _dnorm(degp[...])
    s5 = s5p[0] + s5p[1]
    h5 = _relu(s5 / dmax + g5[...])
    lg = h5[:, :10]
    m = jnp.max(lg, axis=1, keepdims=True)
    e = jnp.exp(lg - m)
    po[...] = e / jnp.sum(e, axis=1, keepdims=True)


def _b5_call(s5p, g5, degp):
    return pl.pallas_call(
        _b5_body,
        grid=(N // BBLK,),
        in_specs=[pl.BlockSpec((2, BBLK, 16), lambda i: (0, i, 0)),
                  _rowsb(16), _degspec()],
        out_specs=_rowsb(10),
        out_shape=_out(10),
    )(s5p, g5, degp)


def _edec_body(zs, zd, wt, wb, b, po):
    po[...] = _mm(zs[...], wt[...]) + _mm(zd[...], wb[...]) + b[...]


def _edec_call(zs, zd, p):
    w = p["edge_fc"]["w"]
    wt = w[:32]
    wb = w[32:]
    b = p["edge_fc"]["b"].reshape(1, -1)
    espec = pl.BlockSpec((EBLK, 32), lambda i: (i, 0))
    return pl.pallas_call(
        _edec_body,
        grid=(E // EBLK,),
        in_specs=[espec, espec, _full(wt.shape), _full(wb.shape),
                  _full(b.shape)],
        out_specs=pl.BlockSpec((EBLK, 16), lambda i: (i, 0)),
        out_shape=jax.ShapeDtypeStruct((E, 16), jnp.float32),
    )(zs, zd, wt, wb, b)


def kernel(x, edge_attr, params, edge_index):
    p = params
    src = edge_index[0].astype(jnp.int32)
    dst = edge_index[1].astype(jnp.int32)
    pad = EPAD - E
    srcp = jnp.concatenate([src, jnp.zeros((pad,), jnp.int32)])
    dstp = jnp.concatenate([dst, jnp.full((pad,), N, jnp.int32)])
    srcw = srcp.reshape(16, WCH, CH)
    dstw = dstp.reshape(16, WCH, CH)
    srcn = srcp.reshape(16, 2, NCH, CH)
    dstn = dstp.reshape(16, 2, NCH, CH)
    zz128 = jnp.zeros((NPAD, 128), jnp.bfloat16)
    zz32 = jnp.zeros((NPAD, 32), jnp.float32)
    zz16 = jnp.zeros((NPAD, 16), jnp.float32)
    ones = jnp.ones((CH, 16), jnp.float32)

    sc = _sc_kernels()
    t1, t2, t3, z, xbar, q, g1lo, g1hi = _ae_call(x, p)
    degp = sc["deg"](dstn, ones, zz16)

    s1lo, s1hi = sc["wide"](g1lo, g1hi, srcw, dstw, zz128)
    u2lo, u2hi = _b1_call(s1lo, s1hi, g1lo, g1hi, t1, degp)
    s2lo, s2hi = sc["wide"](u2lo, u2hi, srcw, dstw, zz128)
    u3lo, u3hi = _b2_call(s2lo, s2hi, u2lo, u2hi, t2, degp, p)
    s3lo, s3hi = sc["wide"](u3lo, u3hi, srcw, dstw, zz128)
    g4 = _b3_call(s3lo, s3hi, u3lo, u3hi, t3, degp, p)
    s4p = sc["narrow32"](g4, srcn, dstn, zz32)
    w5 = jnp.pad(p["proj5"]["w"], ((0, 0), (0, 6)))
    b5 = jnp.pad(p["proj5"]["b"], (0, 6)).reshape(1, -1)
    g5 = _b4_call(s4p, g4, z, degp, w5, b5)
    s5p = sc["narrow16"](g5, srcn, dstn, zz16)
    predict = _b5_call(s5p, g5, degp)

    zs, zd = sc["egather"](z, srcw, dstw)
    pred_edge = _edec_call(zs, zd, p)
    return (xbar, q, predict, z, pred_edge)
